# baseline (speedup 1.0000x reference)
import jax
import jax.numpy as jnp
from jax import lax
from jax.experimental import pallas as pl
from jax.experimental.pallas import tpu as pltpu
from jax.experimental.pallas import tpu_sc as plsc

N = 100000
F = 8
V = 1000
H = 128
NC = 2
NS = 16
NW = NC * NS
G = 16
W = G * F
NG_W = 196
NG = NG_W * NW
NPAD = NG * G
AUG_ROWS = 8192


def _sc_embed(xm_flat, aug):
    mesh = plsc.VectorSubcoreMesh(
        core_axis_name="c", subcore_axis_name="s", num_cores=NC, num_subcores=NS
    )

    def body(xm_hbm, aug_hbm, out_hbm, xmv, idxv, rowsv, outv, aug_sh,
             sxm0, sxm1, sg0, sg1, so0, so1):
        sem_xm = [sxm0, sxm1]
        sem_g = [sg0, sg1]
        sem_o = [so0, so1]
        sid = lax.axis_index("s")
        wid = sid * NC + lax.axis_index("c")

        @pl.when(sid == 0)
        def _stage():
            pltpu.sync_copy(aug_hbm, aug_sh)

        plsc.subcore_barrier()

        fvec = lax.iota(jnp.int32, 16) & (F - 1)
        moff = F * V + fvec
        voff = fvec * V

        def xm_copy(g, b):
            gg = (wid + g * NW) * W
            return pltpu.make_async_copy(
                xm_hbm.at[pl.ds(gg, W)], xmv.at[b], sem_xm[b])

        def gather_copy(b):
            return pltpu.make_async_copy(
                aug_sh.at[idxv.at[b]], rowsv.at[b], sem_g[b])

        def out_copy(g, b):
            gg = (wid + g * NW) * G
            return pltpu.make_async_copy(
                outv.at[b], out_hbm.at[pl.ds(gg, G)], sem_o[b])

        xm_copy(0, 0).start()

        @pl.loop(0, NG_W + 2, step=2)
        def _pair(g0):
            for db in range(2):
                g = g0 + db
                b = db
                nb = 1 - db

                @pl.when(g < NG_W)
                def _front():
                    xm_copy(g, b).wait()

                @pl.when(g + 1 < NG_W)
                def _pf():
                    xm_copy(g + 1, nb).start()

                @pl.when(g < NG_W)
                def _fire():
                    for t in range(W // 16):
                        xi = xmv[b, pl.ds(t * 16, 16)]
                        ml = xi >> 30
                        idxv[b, pl.ds(t * 16, 16)] = jnp.where(
                            ml != 0, moff, (xi & (1024 - 1)) + voff)
                    gather_copy(b).start()

                @pl.when((g >= 1) & (g - 1 < NG_W))
                def _back():
                    @pl.when(g - 3 >= 0)
                    def _wprev():
                        out_copy(g - 3, nb).wait()

                    @pl.loop(0, G, unroll=2)
                    def _node(n):
                        r = n * F
                        for h in range(H // 16):
                            vals = [rowsv[nb, r + f, pl.ds(h * 16, 16)]
                                    for f in range(F)]
                            while len(vals) > 1:
                                vals = [vals[i] + vals[i + 1]
                                        for i in range(0, len(vals), 2)]
                            outv[nb, n, pl.ds(h * 16, 16)] = vals[0]

                    out_copy(g - 1, nb).start()

        out_copy(NG_W - 2, (NG_W - 2) % 2).wait()
        out_copy(NG_W - 1, (NG_W - 1) % 2).wait()

    run = pl.kernel(
        body,
        out_type=jax.ShapeDtypeStruct((NPAD, H), jnp.float32),
        mesh=mesh,
        scratch_types=[
            pltpu.VMEM((2, W), jnp.int32),
            pltpu.VMEM((2, W), jnp.int32),
            pltpu.VMEM((2, W, H), jnp.float32),
            pltpu.VMEM((2, G, H), jnp.float32),
            pltpu.VMEM_SHARED((AUG_ROWS, H), jnp.float32),
            pltpu.SemaphoreType.DMA,
            pltpu.SemaphoreType.DMA,
            pltpu.SemaphoreType.DMA,
            pltpu.SemaphoreType.DMA,
            pltpu.SemaphoreType.DMA,
            pltpu.SemaphoreType.DMA,
        ],
    )
    return run(xm_flat, aug)


def kernel(x, mask, edge_index, edge_type, reliable_masking, tables, mask_emb):
    xm = (x.astype(jnp.int32) | (mask.astype(jnp.int32) << 30)).reshape(N * F)
    xm_flat = jnp.zeros((NG * W,), jnp.int32).at[: N * F].set(xm)
    rm = (jnp.asarray(reliable_masking) != 0).astype(jnp.float32)
    fill = mask_emb * rm
    aug = jnp.concatenate(
        [tables.reshape(F * V, H), fill,
         jnp.zeros((AUG_ROWS - F * V - F, H), jnp.float32)], axis=0)
    return _sc_embed(xm_flat, aug)[:N]


# speedup vs baseline: 1.4187x; 1.4187x over previous
import jax
import jax.numpy as jnp
from jax import lax
from jax.experimental import pallas as pl
from jax.experimental.pallas import tpu as pltpu
from jax.experimental.pallas import tpu_sc as plsc

N = 100000
F = 8
V = 1000
H = 128
NC = 2
NS = 16
NW = NC * NS
G = 16
W = G * F
NG_W = 196
NG = NG_W * NW
NPAD = NG * G
AUG_ROWS = 8192


def _sc_embed(xm_flat, aug):
    mesh = plsc.VectorSubcoreMesh(
        core_axis_name="c", subcore_axis_name="s", num_cores=NC, num_subcores=NS
    )

    def body(xm_hbm, aug_hbm, out_hbm, xmv, idxv, rowsv, outv, aug_sh,
             sxm0, sxm1, sg0, sg1, so0, so1):
        sem_xm = [sxm0, sxm1]
        sem_g = [sg0, sg1]
        sem_o = [so0, so1]
        sid = lax.axis_index("s")
        wid = sid * NC + lax.axis_index("c")

        @pl.when(sid == 0)
        def _stage():
            pltpu.sync_copy(aug_hbm, aug_sh)

        plsc.subcore_barrier()

        fvec = lax.iota(jnp.int32, 16) & (F - 1)
        moff = F * V + fvec
        voff = fvec * V

        def xm_copy(g, b):
            gg = (wid + g * NW) * W
            return pltpu.make_async_copy(
                xm_hbm.at[pl.ds(gg, W)], xmv.at[b], sem_xm[b])

        def gather_copy(b, j):
            return pltpu.make_async_copy(
                aug_sh.at[idxv.at[b, j]],
                rowsv.at[b, pl.ds(j * (W // 2), W // 2)], sem_g[b])

        def out_copy(g, b):
            gg = (wid + g * NW) * G
            return pltpu.make_async_copy(
                outv.at[b], out_hbm.at[pl.ds(gg, G)], sem_o[b])

        xm_copy(0, 0).start()

        @pl.loop(0, NG_W + 2, step=2)
        def _pair(g0):
            for db in range(2):
                g = g0 + db
                b = db
                nb = 1 - db

                @pl.when(g < NG_W)
                def _front():
                    xm_copy(g, b).wait()

                @pl.when(g + 1 < NG_W)
                def _pf():
                    xm_copy(g + 1, nb).start()

                @pl.when(g < NG_W)
                def _fire():
                    for j in range(2):
                        for t in range(W // 32):
                            o = j * (W // 2) + t * 16
                            xi = xmv[b, pl.ds(o, 16)]
                            ml = xi >> 30
                            idxv[b, j, pl.ds(t * 16, 16)] = jnp.where(
                                ml != 0, moff, (xi & (1024 - 1)) + voff)
                        gather_copy(b, j).start()

                @pl.when((g >= 1) & (g - 1 < NG_W))
                def _back():
                    gather_copy(nb, 0).wait()
                    gather_copy(nb, 1).wait()

                    @pl.when(g - 3 >= 0)
                    def _wprev():
                        out_copy(g - 3, nb).wait()

                    @pl.loop(0, G, unroll=2)
                    def _node(n):
                        r = n * F

                        def loads(h):
                            return [rowsv[nb, r + f, pl.ds(h * 16, 16)]
                                    for f in range(F)]

                        cur = loads(0)
                        for h in range(H // 16):
                            nxt = loads(h + 1) if h + 1 < H // 16 else []
                            vals = cur
                            while len(vals) > 1:
                                vals = [vals[i] + vals[i + 1]
                                        for i in range(0, len(vals), 2)]
                            outv[nb, n, pl.ds(h * 16, 16)] = vals[0]
                            cur = nxt

                    out_copy(g - 1, nb).start()

        out_copy(NG_W - 2, (NG_W - 2) % 2).wait()
        out_copy(NG_W - 1, (NG_W - 1) % 2).wait()

    run = pl.kernel(
        body,
        out_type=jax.ShapeDtypeStruct((NPAD, H), jnp.float32),
        mesh=mesh,
        scratch_types=[
            pltpu.VMEM((2, W), jnp.int32),
            pltpu.VMEM((2, 2, W // 2), jnp.int32),
            pltpu.VMEM((2, W, H), jnp.float32),
            pltpu.VMEM((2, G, H), jnp.float32),
            pltpu.VMEM_SHARED((AUG_ROWS, H), jnp.float32),
            pltpu.SemaphoreType.DMA,
            pltpu.SemaphoreType.DMA,
            pltpu.SemaphoreType.DMA,
            pltpu.SemaphoreType.DMA,
            pltpu.SemaphoreType.DMA,
            pltpu.SemaphoreType.DMA,
        ],
    )
    return run(xm_flat, aug)


def kernel(x, mask, edge_index, edge_type, reliable_masking, tables, mask_emb):
    xm = (x.astype(jnp.int32) | (mask.astype(jnp.int32) << 30)).reshape(N * F)
    xm_flat = jnp.zeros((NG * W,), jnp.int32).at[: N * F].set(xm)
    rm = (jnp.asarray(reliable_masking) != 0).astype(jnp.float32)
    fill = mask_emb * rm
    aug = jnp.concatenate(
        [tables.reshape(F * V, H), fill,
         jnp.zeros((AUG_ROWS - F * V - F, H), jnp.float32)], axis=0)
    return _sc_embed(xm_flat, aug)[:N]


# speedup vs baseline: 1.4317x; 1.0092x over previous
import jax
import jax.numpy as jnp
from jax import lax
from jax.experimental import pallas as pl
from jax.experimental.pallas import tpu as pltpu
from jax.experimental.pallas import tpu_sc as plsc

N = 100000
F = 8
V = 1000
H = 128
NC = 2
NS = 16
NW = NC * NS
G = 16
W = G * F
NG_W = 196
NG = NG_W * NW
NPAD = NG * G
AUG_ROWS = 8192
NB = 3


def _sc_embed(xm_flat, aug):
    mesh = plsc.VectorSubcoreMesh(
        core_axis_name="c", subcore_axis_name="s", num_cores=NC, num_subcores=NS
    )

    def body(xm_hbm, aug_hbm, out_hbm, xmv, idxv, rowsv, outv, aug_sh, *sems):
        sem_xm = sems[0:3]
        sem_g = sems[3:6]
        sem_o = sems[6:9]
        sid = lax.axis_index("s")
        wid = sid * NC + lax.axis_index("c")

        @pl.when(sid == 0)
        def _stage():
            pltpu.sync_copy(aug_hbm, aug_sh)

        plsc.subcore_barrier()

        fvec = lax.iota(jnp.int32, 16) & (F - 1)
        moff = F * V + fvec
        voff = fvec * V

        def xm_copy(g, b):
            gg = (wid + g * NW) * W
            return pltpu.make_async_copy(
                xm_hbm.at[pl.ds(gg, W)], xmv.at[b], sem_xm[b])

        def gather_copy(b):
            return pltpu.make_async_copy(
                aug_sh.at[idxv.at[b]], rowsv.at[b], sem_g[b])

        def out_copy(g, b):
            gg = (wid + g * NW) * G
            return pltpu.make_async_copy(
                outv.at[b], out_hbm.at[pl.ds(gg, G)], sem_o[b])

        xm_copy(0, 0).start()
        xm_copy(1, 1).start()

        @pl.loop(0, ((NG_W + 4) // NB) * NB, step=NB)
        def _trip(g0):
            for db in range(NB):
                g = g0 + db
                b = db
                pb = (db + 1) % NB
                fb = (db + 2) % NB

                @pl.when(g < NG_W)
                def _front():
                    xm_copy(g, b).wait()
                    for t in range(W // 16):
                        xi = xmv[b, pl.ds(t * 16, 16)]
                        ml = xi >> 30
                        idxv[b, pl.ds(t * 16, 16)] = jnp.where(
                            ml != 0, moff, (xi & (1024 - 1)) + voff)
                    gather_copy(b).start()

                @pl.when(g + 2 < NG_W)
                def _pf():
                    xm_copy(g + 2, fb).start()

                @pl.when((g >= 2) & (g - 2 < NG_W))
                def _back():
                    gather_copy(pb).wait()

                    @pl.when(g - 5 >= 0)
                    def _wprev():
                        out_copy(g - 5, pb).wait()

                    @pl.loop(0, G, unroll=2)
                    def _node(n):
                        r = n * F

                        def loads(h):
                            return [rowsv[pb, r + f, pl.ds(h * 16, 16)]
                                    for f in range(F)]

                        cur = loads(0)
                        for h in range(H // 16):
                            nxt = loads(h + 1) if h + 1 < H // 16 else []
                            vals = cur
                            while len(vals) > 1:
                                vals = [vals[i] + vals[i + 1]
                                        for i in range(0, len(vals), 2)]
                            outv[pb, n, pl.ds(h * 16, 16)] = vals[0]
                            cur = nxt

                    out_copy(g - 2, pb).start()

        for g in range(NG_W - NB, NG_W):
            out_copy(g, g % NB).wait()

    run = pl.kernel(
        body,
        out_type=jax.ShapeDtypeStruct((NPAD, H), jnp.float32),
        mesh=mesh,
        scratch_types=[
            pltpu.VMEM((NB, W), jnp.int32),
            pltpu.VMEM((NB, W), jnp.int32),
            pltpu.VMEM((NB, W, H), jnp.float32),
            pltpu.VMEM((NB, G, H), jnp.float32),
            pltpu.VMEM_SHARED((AUG_ROWS, H), jnp.float32),
        ] + [pltpu.SemaphoreType.DMA] * 9,
    )
    return run(xm_flat, aug)


def kernel(x, mask, edge_index, edge_type, reliable_masking, tables, mask_emb):
    xm = (x.astype(jnp.int32) | (mask.astype(jnp.int32) << 30)).reshape(N * F)
    xm_flat = jnp.zeros((NG * W,), jnp.int32).at[: N * F].set(xm)
    rm = (jnp.asarray(reliable_masking) != 0).astype(jnp.float32)
    fill = mask_emb * rm
    aug = jnp.concatenate(
        [tables.reshape(F * V, H), fill,
         jnp.zeros((AUG_ROWS - F * V - F, H), jnp.float32)], axis=0)
    return _sc_embed(xm_flat, aug)[:N]
